# baseline (speedup 1.0000x reference)
import jax
import jax.numpy as jnp
from jax.experimental import pallas as pl
from jax.experimental.pallas import tpu as pltpu

B, N, M, D = 8, 2048, 2048, 3


def _chamfer_body(t_ref, s_ref, o_ref, acc_ref):
    b = pl.program_id(0)
    t = t_ref[0]
    s = s_ref[0]
    tn = jnp.sum(t * t, axis=1, keepdims=True)
    sn_col = 0.25 * jnp.sum(s * s, axis=1, keepdims=True)
    sn = jnp.transpose(sn_col, (1, 0))
    prod = jax.lax.dot_general(
        t, s, (((1,), (1,)), ((), ())),
        preferred_element_type=jnp.float32)
    d = prod + tn + sn
    rowmin = jnp.maximum(jnp.min(d, axis=1), 0.0)
    colmin = jnp.maximum(jnp.min(d, axis=0), 0.0)
    s1 = jnp.sum(jnp.sqrt(rowmin))
    s2 = jnp.sum(jnp.sqrt(colmin))

    @pl.when(b == 0)
    def _init():
        acc_ref[0] = 0.0
        acc_ref[1] = 0.0

    acc_ref[0] += s1
    acc_ref[1] += s2

    @pl.when(b == B - 1)
    def _fin():
        c1 = acc_ref[0] / (B * N)
        c2 = acc_ref[1] / (B * M)
        o_ref[0, 0] = (c1 + c2) * 0.5


def kernel(template, source):
    sm = source * -2.0
    out = pl.pallas_call(
        _chamfer_body,
        grid=(B,),
        in_specs=[
            pl.BlockSpec((1, N, D), lambda b: (b, 0, 0)),
            pl.BlockSpec((1, M, D), lambda b: (b, 0, 0)),
        ],
        out_specs=pl.BlockSpec(memory_space=pltpu.SMEM),
        out_shape=jax.ShapeDtypeStruct((1, 1), jnp.float32),
        scratch_shapes=[pltpu.SMEM((2,), jnp.float32)],
    )(template, sm)
    return out[0, 0]
